# baseline (speedup 1.0000x reference)
import functools

import jax
import jax.numpy as jnp
from jax import lax
from jax.experimental import pallas as pl
from jax.experimental.pallas import tpu as pltpu
from jax.experimental.pallas import tpu_sc as plsc

N = 10000
E = 320000
D = 128

NC = 2
NS = 16
NW = NC * NS

CH = 128
NCHT = 160
NCH0 = 122
NCH1 = NCHT - NCH0
E_PAD = NCHT * CH * NS
RPT = -(-N // (NS * CH)) * CH
N_PAD = RPT * NS

BR = 512


def _sc_aggregate_body(feat_hbm, src_hbm, dst_hbm, out_hbm, sidx_a, sidx_b,
                       didx_a, didx_b, rows_a, rows_b, agg, sem_ga, sem_gb,
                       sem_ia, sem_ib):
    c = lax.axis_index("c")
    s = lax.axis_index("s")
    r0 = s * RPT

    def idx_load(ch, si, di, sem):
        pltpu.async_copy(src_hbm.at[s, ch], si, sem)
        pltpu.async_copy(dst_hbm.at[s, ch], di, sem)

    def idx_wait(si, di, sem):
        pltpu.make_async_copy(src_hbm.at[s, 0], si, sem).wait()
        pltpu.make_async_copy(dst_hbm.at[s, 0], di, sem).wait()

    def edge_phase(base, nch):
        idx_load(base, sidx_a, didx_a, sem_ia)
        idx_load(base + 1, sidx_b, didx_b, sem_ib)
        idx_wait(sidx_a, didx_a, sem_ia)
        pltpu.async_copy(feat_hbm.at[sidx_a], rows_a, sem_ga)

        def edge_i(i2, carry):
            i = 2 * i2
            pltpu.make_async_copy(feat_hbm.at[sidx_a], rows_a, sem_ga).wait()
            idx_wait(sidx_b, didx_b, sem_ib)
            pltpu.async_copy(feat_hbm.at[sidx_b], rows_b, sem_gb)
            pltpu.sync_copy(rows_a, agg.at[didx_a], add=True)

            @pl.when(i + 2 < nch)
            def _():
                idx_load(base + i + 2, sidx_a, didx_a, sem_ia)

            pltpu.make_async_copy(feat_hbm.at[sidx_b], rows_b, sem_gb).wait()

            @pl.when(i + 2 < nch)
            def _():
                idx_wait(sidx_a, didx_a, sem_ia)
                pltpu.async_copy(feat_hbm.at[sidx_a], rows_a, sem_ga)

            pltpu.sync_copy(rows_b, agg.at[didx_b], add=True)

            @pl.when(i + 3 < nch)
            def _():
                idx_load(base + i + 3, sidx_b, didx_b, sem_ib)

            return carry

        lax.fori_loop(0, nch // 2, edge_i, 0)

    def init_i(i, carry):
        off = r0 + i * CH
        pltpu.sync_copy(feat_hbm.at[pl.ds(off, CH)], rows_a)
        pltpu.sync_copy(rows_a, agg.at[pl.ds(off, CH)])
        return carry

    lax.fori_loop(0, RPT // CH, init_i, 0)
    plsc.subcore_barrier()

    @pl.when(c == 0)
    def _():
        edge_phase(0, NCH0)

    @pl.when(c == 1)
    def _():
        edge_phase(NCH0, NCH1)

    plsc.subcore_barrier()

    def dump_i(i, carry):
        off = r0 + i * CH
        pltpu.sync_copy(agg.at[pl.ds(off, CH)], rows_a)
        pltpu.sync_copy(rows_a, out_hbm.at[c, pl.ds(off, CH)])
        return carry

    lax.fori_loop(0, RPT // CH, dump_i, 0)


@functools.cache
def _sc_aggregate():
    return pl.kernel(
        _sc_aggregate_body,
        out_type=jax.ShapeDtypeStruct((NC, N_PAD, D), jnp.float32),
        mesh=plsc.VectorSubcoreMesh(core_axis_name="c", subcore_axis_name="s"),
        scratch_types=[
            pltpu.VMEM((CH,), jnp.int32),
            pltpu.VMEM((CH,), jnp.int32),
            pltpu.VMEM((CH,), jnp.int32),
            pltpu.VMEM((CH,), jnp.int32),
            pltpu.VMEM((CH, D), jnp.float32),
            pltpu.VMEM((CH, D), jnp.float32),
            pltpu.VMEM_SHARED((N_PAD, D), jnp.float32),
            pltpu.SemaphoreType.DMA,
            pltpu.SemaphoreType.DMA,
            pltpu.SemaphoreType.DMA,
            pltpu.SemaphoreType.DMA,
        ],
    )


def _tc_body(f_ref, a0_ref, a1_ref, wt_ref, b_ref, o_ref):
    x = a0_ref[0] + a1_ref[0] - f_ref[...]
    y = jnp.dot(x, wt_ref[...], preferred_element_type=jnp.float32)
    o_ref[...] = jnp.maximum(y + b_ref[...], 0.0)


_tc_linear = pl.pallas_call(
    _tc_body,
    grid=(pl.cdiv(N, BR),),
    in_specs=[
        pl.BlockSpec((BR, D), lambda i: (i, 0)),
        pl.BlockSpec((1, BR, D), lambda i: (0, i, 0)),
        pl.BlockSpec((1, BR, D), lambda i: (1, i, 0)),
        pl.BlockSpec((D, D), lambda i: (0, 0)),
        pl.BlockSpec((1, D), lambda i: (0, 0)),
    ],
    out_specs=pl.BlockSpec((BR, D), lambda i: (i, 0)),
    out_shape=jax.ShapeDtypeStruct((N, D), jnp.float32),
)


def kernel(feature, edge_index, W, b):
    pad_e = E_PAD - E
    src = jnp.concatenate(
        [edge_index[0], jnp.zeros((pad_e,), jnp.int32)]).reshape(NS, NCHT, CH)
    dst = jnp.concatenate(
        [edge_index[1], jnp.full((pad_e,), N, jnp.int32)]).reshape(NS, NCHT, CH)
    feat_p = jnp.concatenate(
        [feature, jnp.zeros((N_PAD - N, D), jnp.float32)])
    agg = _sc_aggregate()(feat_p, src, dst)
    return _tc_linear(feature, agg, agg, W.T, b.reshape(1, D))


# speedup vs baseline: 2.4081x; 2.4081x over previous
import functools

import jax
import jax.numpy as jnp
from jax import lax
from jax.experimental import pallas as pl
from jax.experimental.pallas import tpu as pltpu
from jax.experimental.pallas import tpu_sc as plsc

N = 10000
E = 320000
D = 128

NC = 2
NS = 16
NW = NC * NS

CH = 128
NCH = 80
EPW = NCH * CH
E_PAD = EPW * NW
RPT = -(-N // (NS * CH)) * CH
N_PAD = RPT * NS

BR = 512


def _sc_aggregate_body(feat_hbm, src_hbm, dst_hbm, out_hbm, sidx_a, sidx_b,
                       didx_a, didx_b, rows_a, rows_b, agg, sem_ga, sem_gb,
                       sem_ia, sem_ib):
    c = lax.axis_index("c")
    s = lax.axis_index("s")
    w = s * NC + c
    r0 = s * RPT

    pltpu.async_copy(src_hbm.at[w, 0], sidx_a, sem_ia)
    pltpu.async_copy(dst_hbm.at[w, 0], didx_a, sem_ia)
    pltpu.async_copy(src_hbm.at[w, 1], sidx_b, sem_ib)
    pltpu.async_copy(dst_hbm.at[w, 1], didx_b, sem_ib)

    def init_i(i, carry):
        off = r0 + i * CH
        pltpu.sync_copy(feat_hbm.at[pl.ds(off, CH)], rows_a)
        pltpu.sync_copy(rows_a, agg.at[pl.ds(off, CH)])
        return carry

    lax.fori_loop(0, RPT // CH, init_i, 0)
    plsc.subcore_barrier()

    pltpu.make_async_copy(src_hbm.at[w, 0], sidx_a, sem_ia).wait()
    pltpu.make_async_copy(dst_hbm.at[w, 0], didx_a, sem_ia).wait()
    pltpu.async_copy(feat_hbm.at[sidx_a], rows_a, sem_ga)

    def edge_i(i2, carry):
        i = 2 * i2
        pltpu.make_async_copy(feat_hbm.at[sidx_a], rows_a, sem_ga).wait()
        pltpu.make_async_copy(src_hbm.at[w, 0], sidx_b, sem_ib).wait()
        pltpu.make_async_copy(dst_hbm.at[w, 0], didx_b, sem_ib).wait()
        pltpu.async_copy(feat_hbm.at[sidx_b], rows_b, sem_gb)
        pltpu.sync_copy(rows_a, agg.at[didx_a], add=True)

        @pl.when(i + 2 < NCH)
        def _():
            pltpu.async_copy(src_hbm.at[w, i + 2], sidx_a, sem_ia)
            pltpu.async_copy(dst_hbm.at[w, i + 2], didx_a, sem_ia)

        pltpu.make_async_copy(feat_hbm.at[sidx_b], rows_b, sem_gb).wait()

        @pl.when(i + 2 < NCH)
        def _():
            pltpu.make_async_copy(src_hbm.at[w, 0], sidx_a, sem_ia).wait()
            pltpu.make_async_copy(dst_hbm.at[w, 0], didx_a, sem_ia).wait()
            pltpu.async_copy(feat_hbm.at[sidx_a], rows_a, sem_ga)

        pltpu.sync_copy(rows_b, agg.at[didx_b], add=True)

        @pl.when(i + 3 < NCH)
        def _():
            pltpu.async_copy(src_hbm.at[w, i + 3], sidx_b, sem_ib)
            pltpu.async_copy(dst_hbm.at[w, i + 3], didx_b, sem_ib)

        return carry

    lax.fori_loop(0, NCH // 2, edge_i, 0)
    plsc.subcore_barrier()

    def dump_i(i, carry):
        off = r0 + i * CH
        pltpu.sync_copy(agg.at[pl.ds(off, CH)], rows_a)
        pltpu.sync_copy(rows_a, out_hbm.at[c, pl.ds(off, CH)])
        return carry

    lax.fori_loop(0, RPT // CH, dump_i, 0)


@functools.cache
def _sc_aggregate():
    return pl.kernel(
        _sc_aggregate_body,
        out_type=jax.ShapeDtypeStruct((NC, N_PAD, D), jnp.float32),
        mesh=plsc.VectorSubcoreMesh(core_axis_name="c", subcore_axis_name="s"),
        scratch_types=[
            pltpu.VMEM((CH,), jnp.int32),
            pltpu.VMEM((CH,), jnp.int32),
            pltpu.VMEM((CH,), jnp.int32),
            pltpu.VMEM((CH,), jnp.int32),
            pltpu.VMEM((CH, D), jnp.float32),
            pltpu.VMEM((CH, D), jnp.float32),
            pltpu.VMEM_SHARED((N_PAD, D), jnp.float32),
            pltpu.SemaphoreType.DMA,
            pltpu.SemaphoreType.DMA,
            pltpu.SemaphoreType.DMA,
            pltpu.SemaphoreType.DMA,
        ],
    )


def _tc_body(f_ref, a0_ref, a1_ref, wt_ref, b_ref, o_ref):
    x = a0_ref[0] + a1_ref[0] - f_ref[...]
    y = jnp.dot(x, wt_ref[...], preferred_element_type=jnp.float32)
    o_ref[...] = jnp.maximum(y + b_ref[...], 0.0)


_tc_linear = pl.pallas_call(
    _tc_body,
    grid=(pl.cdiv(N, BR),),
    in_specs=[
        pl.BlockSpec((BR, D), lambda i: (i, 0)),
        pl.BlockSpec((1, BR, D), lambda i: (0, i, 0)),
        pl.BlockSpec((1, BR, D), lambda i: (1, i, 0)),
        pl.BlockSpec((D, D), lambda i: (0, 0)),
        pl.BlockSpec((1, D), lambda i: (0, 0)),
    ],
    out_specs=pl.BlockSpec((BR, D), lambda i: (i, 0)),
    out_shape=jax.ShapeDtypeStruct((N, D), jnp.float32),
)


def kernel(feature, edge_index, W, b):
    pad_e = E_PAD - E
    pad_src = jnp.arange(pad_e, dtype=jnp.int32) % N
    src = jnp.concatenate(
        [edge_index[0], pad_src]).reshape(NW, NCH, CH)
    pad_dst = N + jnp.arange(pad_e, dtype=jnp.int32) % (N_PAD - N)
    dst = jnp.concatenate(
        [edge_index[1], pad_dst]).reshape(NW, NCH, CH)
    feat_p = jnp.concatenate(
        [feature, jnp.zeros((N_PAD - N, D), jnp.float32)])
    agg = _sc_aggregate()(feat_p, src, dst)
    return _tc_linear(feature, agg, agg, W.T, b.reshape(1, D))
